# baseline (speedup 1.0000x reference)
import functools

import jax
import jax.numpy as jnp
from jax import lax
from jax.experimental import pallas as pl
from jax.experimental.pallas import tpu as pltpu
from jax.experimental.pallas import tpu_sc as plsc

N = 4096
D = 64
KEEP = 21
BLOCK = 256
NUM_WORKERS = 32
ROWS_PER_W = N // NUM_WORKERS
LANES = 16


def _sim_kernel(x_ref, w0_ref, b0_ref, w1_ref, b1_ref, out_ref,
                h_ref, ht_ref):
    i = pl.program_id(0)

    @pl.when(i == 0)
    def _embed():
        x = x_ref[...]
        h = jnp.dot(x, w0_ref[...].T, preferred_element_type=jnp.float32)
        h = jnp.maximum(h + b0_ref[...], 0.0)
        h = jnp.dot(h, w1_ref[...].T, preferred_element_type=jnp.float32)
        h = h + b1_ref[...]
        nrm = jnp.sqrt(jnp.sum(h * h, axis=1, keepdims=True))
        h = h / jnp.maximum(nrm, 1e-12)
        h_ref[...] = h
        ht_ref[...] = h.T

    hb = h_ref[pl.ds(i * BLOCK, BLOCK), :]
    out_ref[...] = jnp.dot(hb, ht_ref[...], preferred_element_type=jnp.float32)


def _similarity(x, W0, b0, W1, b1):
    grid = (N // BLOCK,)
    return pl.pallas_call(
        _sim_kernel,
        grid=grid,
        in_specs=[
            pl.BlockSpec((N, D), lambda i: (0, 0)),
            pl.BlockSpec((D, D), lambda i: (0, 0)),
            pl.BlockSpec((1, D), lambda i: (0, 0)),
            pl.BlockSpec((D, D), lambda i: (0, 0)),
            pl.BlockSpec((1, D), lambda i: (0, 0)),
        ],
        out_specs=pl.BlockSpec((BLOCK, N), lambda i: (i, 0)),
        out_shape=jax.ShapeDtypeStruct((N, N), jnp.float32),
        scratch_shapes=[
            pltpu.VMEM((N, D), jnp.float32),
            pltpu.VMEM((D, N), jnp.float32),
        ],
        compiler_params=pltpu.CompilerParams(
            dimension_semantics=("arbitrary",),
        ),
    )(x, W0, b0, W1, b1)


def _sc_topk_body(sim_hbm, out_hbm, row_v, orow_v):
    c = lax.axis_index("c")
    s = lax.axis_index("s")
    wid = c * 16 + s
    lane = lax.iota(jnp.int32, LANES)

    def do_row(r, carry):
        row = wid * ROWS_PER_W + r
        pltpu.sync_copy(sim_hbm.at[row], row_v)

        def _sort16(v):
            _, sv = plsc.sort_key_val(v, v)
            return sv

        def chunk_step(j, bufs):
            b1, b2 = bufs
            off = pl.multiple_of(j * LANES, LANES)
            cvec = row_v[pl.ds(off, LANES)]
            cs = lax.rev(_sort16(cvec), (0,))
            h = jnp.maximum(b1, cs)
            l = jnp.minimum(b1, cs)
            b1 = _sort16(h)
            ls = lax.rev(_sort16(l), (0,))
            b2 = _sort16(jnp.maximum(b2, ls))
            return (b1, b2)

        neg = jnp.full((LANES,), -2.0, jnp.float32)
        _, b2 = lax.fori_loop(0, N // LANES, chunk_step, (neg, neg))
        t = jnp.max(jnp.where(lane == 11, b2, -2.0))
        t = jnp.maximum(t, 0.0)
        tb = jnp.full((LANES,), t)

        def wchunk(j, carry2):
            off = pl.multiple_of(j * LANES, LANES)
            cvec = row_v[pl.ds(off, LANES)]
            orow_v[pl.ds(off, LANES)] = jnp.where(cvec >= tb, cvec, 0.0)
            return carry2

        lax.fori_loop(0, N // LANES, wchunk, 0)
        pltpu.sync_copy(orow_v, out_hbm.at[row])
        return carry

    lax.fori_loop(0, ROWS_PER_W, do_row, 0)


def _sc_topk(sim):
    mesh = plsc.VectorSubcoreMesh(core_axis_name="c", subcore_axis_name="s")
    return pl.kernel(
        _sc_topk_body,
        out_type=jax.ShapeDtypeStruct((N, N), jnp.float32),
        mesh=mesh,
        scratch_types=[
            pltpu.VMEM((N,), jnp.float32),
            pltpu.VMEM((N,), jnp.float32),
        ],
        compiler_params=pltpu.CompilerParams(needs_layout_passes=False),
    )(sim)


def kernel(x, W0, b0, W1, b1):
    x = x.astype(jnp.float32)
    b0 = b0.reshape(1, D).astype(jnp.float32)
    b1 = b1.reshape(1, D).astype(jnp.float32)
    sim = _similarity(x, W0.astype(jnp.float32), b0,
                      W1.astype(jnp.float32), b1)
    return _sc_topk(sim)
